# baseline (speedup 1.0000x reference)
import functools

import jax
import jax.numpy as jnp
from jax.experimental import pallas as pl
from jax.experimental.pallas import tpu as pltpu

N_EXPERTS = 64
TOP_K = 8
BR = 1024


def _router_block(xa_ref, xb_ref, wta_ref, wtb_ref, scale_ref,
                  rm_ref, idx_ref, lg_ref):
    logits = lg_ref[...]
    scale = scale_ref[0]

    lane_f = jax.lax.broadcasted_iota(
        jnp.int32, logits.shape, 1).astype(jnp.float32)
    neg_inf = jnp.float32(-jnp.inf)
    cur = logits
    m1 = None
    idxs = []
    for j in range(TOP_K):
        m = jnp.max(cur, axis=-1, keepdims=True)
        if j == 0:
            m1 = m
        hit = cur == m
        idxs.append(jnp.min(jnp.where(hit, lane_f, 64.0), axis=-1,
                            keepdims=True))
        cur = jnp.where(lane_f == idxs[-1], neg_inf, cur)

    sel = cur == neg_inf
    e = jnp.where(sel, jnp.exp((logits - m1) * scale), 0.0)
    inv = 1.0 / jnp.sum(e, axis=-1, keepdims=True)
    rm_ref[...] = e * inv
    idx_ref[...] = jnp.concatenate(idxs, axis=-1).astype(jnp.int32)

    lg_ref[...] = (
        jnp.dot(xa_ref[...], wta_ref[...], preferred_element_type=jnp.float32)
        + jnp.dot(xb_ref[...], wtb_ref[...], preferred_element_type=jnp.float32))


@jax.jit
def kernel(x, W, temperature):
    n_rows = x.shape[0]
    kdim = x.shape[1]
    kh = kdim // 2
    nb = n_rows // BR
    wt = W.T
    scale = (1.0 / (jnp.abs(temperature) + 1e-5)).reshape(1).astype(jnp.float32)
    rm, idx = pl.pallas_call(
        _router_block,
        grid=(nb + 1,),
        in_specs=[
            pl.BlockSpec((BR, kh), lambda i: (jnp.minimum(i, nb - 1), 0)),
            pl.BlockSpec((BR, kh), lambda i: (jnp.minimum(i, nb - 1), 1)),
            pl.BlockSpec((kh, N_EXPERTS), lambda i: (0, 0)),
            pl.BlockSpec((kh, N_EXPERTS), lambda i: (1, 0)),
            pl.BlockSpec(memory_space=pltpu.SMEM),
        ],
        out_specs=[
            pl.BlockSpec((BR, N_EXPERTS),
                         lambda i: (jnp.maximum(i - 1, 0), 0)),
            pl.BlockSpec((BR, TOP_K),
                         lambda i: (jnp.maximum(i - 1, 0), 0)),
        ],
        out_shape=[
            jax.ShapeDtypeStruct((n_rows, N_EXPERTS), jnp.float32),
            jax.ShapeDtypeStruct((n_rows, TOP_K), jnp.int32),
        ],
        scratch_shapes=[pltpu.VMEM((BR, N_EXPERTS), jnp.float32)],
    )(x, x, wt, wt, scale)
    return (rm, idx)


# speedup vs baseline: 1.0001x; 1.0001x over previous
import functools

import jax
import jax.numpy as jnp
from jax.experimental import pallas as pl
from jax.experimental.pallas import tpu as pltpu

N_EXPERTS = 64
TOP_K = 8
BR = 1024


def _router_block(x_ref, wt_ref, scale_ref, rm_ref, idx_ref, lg_ref):
    logits = lg_ref[...]
    scale = scale_ref[0]

    lane_f = jax.lax.broadcasted_iota(
        jnp.int32, logits.shape, 1).astype(jnp.float32)
    neg_inf = jnp.float32(-jnp.inf)
    cur = logits
    m1 = None
    idxs = []
    for j in range(TOP_K):
        m = jnp.max(cur, axis=-1, keepdims=True)
        if j == 0:
            m1 = m
        hit = cur == m
        idxs.append(jnp.min(jnp.where(hit, lane_f, 64.0), axis=-1,
                            keepdims=True))
        cur = jnp.where(lane_f == idxs[-1], neg_inf, cur)

    sel = cur == neg_inf
    e = jnp.where(sel, jnp.exp((logits - m1) * scale), 0.0)
    inv = 1.0 / jnp.sum(e, axis=-1, keepdims=True)
    rm_ref[...] = e * inv
    idx_ref[...] = jnp.concatenate(idxs, axis=-1).astype(jnp.int32)

    lg_ref[...] = jnp.dot(x_ref[...], wt_ref[...],
                          preferred_element_type=jnp.float32)


@jax.jit
def kernel(x, W, temperature):
    n_rows = x.shape[0]
    nb = n_rows // BR
    wt = W.T
    scale = (1.0 / (jnp.abs(temperature) + 1e-5)).reshape(1).astype(jnp.float32)
    rm, idx = pl.pallas_call(
        _router_block,
        grid=(nb + 1,),
        in_specs=[
            pl.BlockSpec((BR, x.shape[1]),
                         lambda i: (jnp.minimum(i, nb - 1), 0)),
            pl.BlockSpec((x.shape[1], N_EXPERTS), lambda i: (0, 0)),
            pl.BlockSpec(memory_space=pltpu.SMEM),
        ],
        out_specs=[
            pl.BlockSpec((BR, N_EXPERTS),
                         lambda i: (jnp.maximum(i - 1, 0), 0)),
            pl.BlockSpec((BR, TOP_K),
                         lambda i: (jnp.maximum(i - 1, 0), 0)),
        ],
        out_shape=[
            jax.ShapeDtypeStruct((n_rows, N_EXPERTS), jnp.float32),
            jax.ShapeDtypeStruct((n_rows, TOP_K), jnp.int32),
        ],
        scratch_shapes=[pltpu.VMEM((BR, N_EXPERTS), jnp.float32)],
    )(x, wt, scale)
    return (rm, idx)
